# baseline (speedup 1.0000x reference)
import functools

import jax
import jax.numpy as jnp
from jax import lax
from jax.experimental import pallas as pl
from jax.experimental.pallas import tpu as pltpu
from jax.experimental.pallas import tpu_sc as plsc

N = 10000
E = 320000
IN_DIM = 128
HID = 64
OUT = 16
EPS = 1e-5

NP = 10240
D1 = HID
D2 = OUT
NC = 2
NS = 16
NWORK = NC * NS
E_PAD = 327680
TPT = E_PAD // NWORK
ZCH = 128
ROWS_PER_TILE = NP // NS
ZB = ROWS_PER_TILE // ZCH
DD = 16


def _make_sc_aggregate(d, tb, nbuf, with_deg):
    mesh = plsc.VectorSubcoreMesh(core_axis_name="c", subcore_axis_name="s")
    nt = TPT // tb
    n_iter = nt // nbuf
    assert nt % nbuf == 0

    def body(*refs):
        it = iter(refs)
        p_hbm = next(it); src_hbm = next(it); dst_hbm = next(it)
        z_hbm = next(it)
        if with_deg:
            zd_hbm = next(it); ones_hbm = next(it)
        out_hbm = next(it)
        if with_deg:
            deg_hbm = next(it)
        src_v = next(it); dst_v = next(it)
        rows = [next(it) for _ in range(nbuf)]
        zbuf_v = next(it)
        if with_deg:
            ones_v = next(it); dbuf_v = next(it)
        acc_sh = next(it)
        if with_deg:
            deg_sh = next(it)
        gsem = [next(it) for _ in range(nbuf)]
        ssem = [next(it) for _ in range(nbuf)]
        if with_deg:
            dsem = [next(it) for _ in range(nbuf)]

        cid = lax.axis_index("c")
        sid = lax.axis_index("s")
        tile = cid * NS + sid

        pltpu.sync_copy(z_hbm, zbuf_v)
        if with_deg:
            pltpu.sync_copy(zd_hbm, dbuf_v)
            pltpu.sync_copy(ones_hbm, ones_v)
        for j in range(ZB):
            r0 = sid * ROWS_PER_TILE + j * ZCH
            pltpu.sync_copy(zbuf_v, acc_sh.at[pl.ds(r0, ZCH)])
            if with_deg:
                pltpu.sync_copy(dbuf_v, deg_sh.at[pl.ds(r0, ZCH)])
        plsc.subcore_barrier()

        base = tile * TPT
        pltpu.sync_copy(src_hbm.at[pl.ds(base, TPT)], src_v)
        pltpu.sync_copy(dst_hbm.at[pl.ds(base, TPT)], dst_v)

        def sidx(g):
            return src_v.at[pl.ds(g * tb, tb)]

        def didx(g):
            return dst_v.at[pl.ds(g * tb, tb)]

        for k in range(nbuf):
            pltpu.async_copy(p_hbm.at[sidx(k)], rows[k], gsem[k])

        def step(i, carry):
            for k in range(nbuf):
                g = nbuf * i + k
                pltpu.make_async_copy(p_hbm.at[sidx(g)], rows[k],
                                      gsem[k]).wait()
                pltpu.async_copy(rows[k], acc_sh.at[didx(g)], ssem[k],
                                 add=True)
                if with_deg:
                    pltpu.async_copy(ones_v, deg_sh.at[didx(g)], dsem[k],
                                     add=True)

                @pl.when(i < n_iter - 1)
                def _():
                    pltpu.make_async_copy(rows[k], acc_sh.at[didx(g)],
                                          ssem[k]).wait()
                    if with_deg:
                        pltpu.make_async_copy(ones_v, deg_sh.at[didx(g)],
                                              dsem[k]).wait()
                    pltpu.async_copy(p_hbm.at[sidx(g + nbuf)], rows[k],
                                     gsem[k])

            return carry

        lax.fori_loop(0, n_iter, step, 0)
        for k in range(nbuf):
            g = nt - nbuf + k
            pltpu.make_async_copy(rows[k], acc_sh.at[didx(g)],
                                  ssem[k]).wait()
            if with_deg:
                pltpu.make_async_copy(ones_v, deg_sh.at[didx(g)],
                                      dsem[k]).wait()
        plsc.subcore_barrier()

        for j in range(ZB):
            r0 = sid * ROWS_PER_TILE + j * ZCH
            pltpu.sync_copy(acc_sh.at[pl.ds(r0, ZCH)], zbuf_v)
            pltpu.sync_copy(zbuf_v, out_hbm.at[cid, pl.ds(r0, ZCH)])
            if with_deg:
                pltpu.sync_copy(deg_sh.at[pl.ds(r0, ZCH)], dbuf_v)
                pltpu.sync_copy(dbuf_v, deg_hbm.at[cid, pl.ds(r0, ZCH)])

    out_type = [jax.ShapeDtypeStruct((NC, NP, d), jnp.float32)]
    if with_deg:
        out_type.append(jax.ShapeDtypeStruct((NC, NP, DD), jnp.float32))
    scratch = [
        pltpu.VMEM((TPT,), jnp.int32),
        pltpu.VMEM((TPT,), jnp.int32),
    ]
    scratch += [pltpu.VMEM((tb, d), jnp.float32) for _ in range(nbuf)]
    scratch.append(pltpu.VMEM((ZCH, d), jnp.float32))
    if with_deg:
        scratch.append(pltpu.VMEM((tb, DD), jnp.float32))
        scratch.append(pltpu.VMEM((ZCH, DD), jnp.float32))
    scratch.append(pltpu.VMEM_SHARED((NP, d), jnp.float32))
    if with_deg:
        scratch.append(pltpu.VMEM_SHARED((NP, DD), jnp.float32))
    n_sems = nbuf * (3 if with_deg else 2)
    scratch += [pltpu.SemaphoreType.DMA for _ in range(n_sems)]

    return functools.partial(
        pl.kernel,
        out_type=out_type,
        mesh=mesh,
        scratch_types=scratch,
        compiler_params=pltpu.CompilerParams(use_tc_tiling_on_sc=False),
    )(body)


_sc_agg1 = _make_sc_aggregate(D1, 256, 2, True)
_sc_agg2 = _make_sc_aggregate(D2, 512, 4, False)

_DN = (((1,), (1,)), ((), ()))


NG = E // 128
NGP = E_PAD // 128


def _tc_pre(x_ref, wl_ref, wr_ref, ei_ref, p1_ref, r1_ref, src_ref, dst_ref):
    x = x_ref[...]
    xw = lax.dot_general(x, wl_ref[...], _DN, preferred_element_type=jnp.float32)
    xr = lax.dot_general(x, wr_ref[...], _DN, preferred_element_type=jnp.float32)
    pad0 = jnp.zeros((NP - N, HID), jnp.float32)
    r1_ref[...] = jnp.concatenate([xr, pad0], axis=0)
    p1_ref[...] = jnp.concatenate([xw, pad0], axis=0)
    i2 = (lax.broadcasted_iota(jnp.int32, (NGP - NG, 128), 0) * 128
          + lax.broadcasted_iota(jnp.int32, (NGP - NG, 128), 1))
    fill = N + lax.rem(i2, jnp.int32(NP - N))
    src_ref[...] = jnp.concatenate(
        [ei_ref[0].reshape(NG, 128), fill], axis=0).reshape(E_PAD)
    dst_ref[...] = jnp.concatenate(
        [ei_ref[1].reshape(NG, 128), fill], axis=0).reshape(E_PAD)


def _tc_mid(part_ref, degp_ref, r1_ref, bl1_ref, gamma_ref, beta_ref,
            wl2_ref, wr2_ref, p2_ref, r2_ref, invdeg_ref):
    acc = part_ref[0] + part_ref[1]
    deg = degp_ref[0, :, 0:1] + degp_ref[1, :, 0:1]
    invdeg = 1.0 / jnp.maximum(deg, 1.0)
    invdeg_ref[...] = invdeg
    h = acc * invdeg + bl1_ref[...] + r1_ref[...]
    rows = lax.broadcasted_iota(jnp.int32, (NP, 1), 0)
    mask = rows < N
    hm = jnp.where(mask, h, 0.0)
    mean = jnp.sum(hm, axis=0, keepdims=True) * (1.0 / N)
    cm = jnp.where(mask, h - mean, 0.0)
    var = jnp.sum(cm * cm, axis=0, keepdims=True) * (1.0 / N)
    hn = (h - mean) * lax.rsqrt(var + EPS) * gamma_ref[...] + beta_ref[...]
    hr = jnp.maximum(hn, 0.0)
    p2_ref[...] = lax.dot_general(hr, wl2_ref[...], _DN,
                                  preferred_element_type=jnp.float32)
    r2_ref[...] = lax.dot_general(hr, wr2_ref[...], _DN,
                                  preferred_element_type=jnp.float32)


def _tc_post(part2_ref, r2_ref, invdeg_ref, bl2_ref, out_ref):
    acc = part2_ref[0] + part2_ref[1]
    val = acc * invdeg_ref[...] + bl2_ref[...] + r2_ref[...]
    out_ref[...] = val[:N]


def kernel(x, edge_index, Wl1, bl1, Wr1, gamma, beta, Wl2, bl2, Wr2):
    ei = edge_index.astype(jnp.int32)
    z1 = jnp.zeros((ZCH, D1), jnp.float32)
    z2 = jnp.zeros((ZCH, D2), jnp.float32)
    zd = jnp.zeros((ZCH, DD), jnp.float32)
    onesb = jnp.zeros((256, DD), jnp.float32).at[:, 0].set(1.0)
    bl1r = bl1.reshape(1, HID)
    gammar = gamma.reshape(1, HID)
    betar = beta.reshape(1, HID)
    bl2r = bl2.reshape(1, OUT)

    p1ext, r1, src2d, dst2d = pl.pallas_call(
        _tc_pre,
        out_shape=[jax.ShapeDtypeStruct((NP, D1), jnp.float32),
                   jax.ShapeDtypeStruct((NP, HID), jnp.float32),
                   jax.ShapeDtypeStruct((E_PAD,), jnp.int32),
                   jax.ShapeDtypeStruct((E_PAD,), jnp.int32)],
    )(x, Wl1, Wr1, ei)

    part1, degp = _sc_agg1(p1ext, src2d, dst2d, z1, zd, onesb)

    p2, r2, invdeg = pl.pallas_call(
        _tc_mid,
        out_shape=[jax.ShapeDtypeStruct((NP, D2), jnp.float32),
                   jax.ShapeDtypeStruct((NP, D2), jnp.float32),
                   jax.ShapeDtypeStruct((NP, 1), jnp.float32)],
    )(part1, degp, r1, bl1r, gammar, betar, Wl2, Wr2)

    (part2,) = _sc_agg2(p2, src2d, dst2d, z2)

    out = pl.pallas_call(
        _tc_post,
        out_shape=jax.ShapeDtypeStruct((N, D2), jnp.float32),
    )(part2, r2, invdeg, bl2r)
    return out


# speedup vs baseline: 1.0248x; 1.0248x over previous
import functools

import jax
import jax.numpy as jnp
from jax import lax
from jax.experimental import pallas as pl
from jax.experimental.pallas import tpu as pltpu
from jax.experimental.pallas import tpu_sc as plsc

N = 10000
E = 320000
IN_DIM = 128
HID = 64
OUT = 16
EPS = 1e-5

NP = 10240
D1 = HID
D2 = OUT
NC = 2
NS = 16
NWORK = NC * NS
E_PAD = 327680
TPT = E_PAD // NWORK
ZCH = 128
ROWS_PER_TILE = NP // NS
ZB = ROWS_PER_TILE // ZCH
DD = 16


def _make_sc_aggregate(d, tb, nbuf, with_deg):
    mesh = plsc.VectorSubcoreMesh(core_axis_name="c", subcore_axis_name="s")
    nt = TPT // tb
    n_iter = nt // nbuf
    assert nt % nbuf == 0

    def body(*refs):
        it = iter(refs)
        p_hbm = next(it); src_hbm = next(it); dst_hbm = next(it)
        z_hbm = next(it)
        if with_deg:
            zd_hbm = next(it); ones_hbm = next(it)
        out_hbm = next(it)
        if with_deg:
            deg_hbm = next(it)
        src_v = next(it); dst_v = next(it)
        rows = [next(it) for _ in range(nbuf)]
        zbuf_v = next(it)
        if with_deg:
            ones_v = next(it); dbuf_v = next(it)
        acc_sh = next(it)
        if with_deg:
            deg_sh = next(it)
        gsem = [next(it) for _ in range(nbuf)]
        ssem = [next(it) for _ in range(nbuf)]
        if with_deg:
            dsem = [next(it) for _ in range(nbuf)]

        cid = lax.axis_index("c")
        sid = lax.axis_index("s")
        tile = cid * NS + sid

        pltpu.sync_copy(z_hbm, zbuf_v)
        if with_deg:
            pltpu.sync_copy(zd_hbm, dbuf_v)
            pltpu.sync_copy(ones_hbm, ones_v)
        for j in range(ZB):
            r0 = sid * ROWS_PER_TILE + j * ZCH
            pltpu.sync_copy(zbuf_v, acc_sh.at[pl.ds(r0, ZCH)])
            if with_deg:
                pltpu.sync_copy(dbuf_v, deg_sh.at[pl.ds(r0, ZCH)])
        plsc.subcore_barrier()

        base = tile * TPT
        pltpu.sync_copy(src_hbm.at[pl.ds(base, TPT)], src_v)
        pltpu.sync_copy(dst_hbm.at[pl.ds(base, TPT)], dst_v)

        def sidx(g):
            return src_v.at[pl.ds(g * tb, tb)]

        def didx(g):
            return dst_v.at[pl.ds(g * tb, tb)]

        for k in range(nbuf):
            pltpu.async_copy(p_hbm.at[sidx(k)], rows[k], gsem[k])

        def step(i, carry):
            for k in range(nbuf):
                g = nbuf * i + k
                pltpu.make_async_copy(p_hbm.at[sidx(g)], rows[k],
                                      gsem[k]).wait()
                pltpu.async_copy(rows[k], acc_sh.at[didx(g)], ssem[k],
                                 add=True)
                if with_deg:
                    pltpu.async_copy(ones_v, deg_sh.at[didx(g)], dsem[k],
                                     add=True)

                @pl.when(i < n_iter - 1)
                def _():
                    pltpu.make_async_copy(rows[k], acc_sh.at[didx(g)],
                                          ssem[k]).wait()
                    if with_deg:
                        pltpu.make_async_copy(ones_v, deg_sh.at[didx(g)],
                                              dsem[k]).wait()
                    pltpu.async_copy(p_hbm.at[sidx(g + nbuf)], rows[k],
                                     gsem[k])

            return carry

        lax.fori_loop(0, n_iter, step, 0)
        for k in range(nbuf):
            g = nt - nbuf + k
            pltpu.make_async_copy(rows[k], acc_sh.at[didx(g)],
                                  ssem[k]).wait()
            if with_deg:
                pltpu.make_async_copy(ones_v, deg_sh.at[didx(g)],
                                      dsem[k]).wait()
        plsc.subcore_barrier()

        for j in range(ZB):
            r0 = sid * ROWS_PER_TILE + j * ZCH
            pltpu.sync_copy(acc_sh.at[pl.ds(r0, ZCH)], zbuf_v)
            pltpu.sync_copy(zbuf_v, out_hbm.at[cid, pl.ds(r0, ZCH)])
            if with_deg:
                pltpu.sync_copy(deg_sh.at[pl.ds(r0, ZCH)], dbuf_v)
                pltpu.sync_copy(dbuf_v, deg_hbm.at[cid, pl.ds(r0, ZCH)])

    out_type = [jax.ShapeDtypeStruct((NC, NP, d), jnp.float32)]
    if with_deg:
        out_type.append(jax.ShapeDtypeStruct((NC, NP, DD), jnp.float32))
    scratch = [
        pltpu.VMEM((TPT,), jnp.int32),
        pltpu.VMEM((TPT,), jnp.int32),
    ]
    scratch += [pltpu.VMEM((tb, d), jnp.float32) for _ in range(nbuf)]
    scratch.append(pltpu.VMEM((ZCH, d), jnp.float32))
    if with_deg:
        scratch.append(pltpu.VMEM((tb, DD), jnp.float32))
        scratch.append(pltpu.VMEM((ZCH, DD), jnp.float32))
    scratch.append(pltpu.VMEM_SHARED((NP, d), jnp.float32))
    if with_deg:
        scratch.append(pltpu.VMEM_SHARED((NP, DD), jnp.float32))
    n_sems = nbuf * (3 if with_deg else 2)
    scratch += [pltpu.SemaphoreType.DMA for _ in range(n_sems)]

    return functools.partial(
        pl.kernel,
        out_type=out_type,
        mesh=mesh,
        scratch_types=scratch,
        compiler_params=pltpu.CompilerParams(use_tc_tiling_on_sc=False),
    )(body)


_sc_agg1 = _make_sc_aggregate(D1, 128, 5, True)
_sc_agg2 = _make_sc_aggregate(D2, 128, 8, False)

_DN = (((1,), (1,)), ((), ()))


NG = E // 128
NGP = E_PAD // 128


def _tc_pre(x_ref, wl_ref, wr_ref, ei_ref, p1_ref, r1_ref, src_ref, dst_ref):
    x = x_ref[...]
    xw = lax.dot_general(x, wl_ref[...], _DN, preferred_element_type=jnp.float32)
    xr = lax.dot_general(x, wr_ref[...], _DN, preferred_element_type=jnp.float32)
    pad0 = jnp.zeros((NP - N, HID), jnp.float32)
    r1_ref[...] = jnp.concatenate([xr, pad0], axis=0)
    p1_ref[...] = jnp.concatenate([xw, pad0], axis=0)
    i2 = (lax.broadcasted_iota(jnp.int32, (NGP - NG, 128), 0) * 128
          + lax.broadcasted_iota(jnp.int32, (NGP - NG, 128), 1))
    fill = N + lax.rem(i2, jnp.int32(NP - N))
    src_ref[...] = jnp.concatenate(
        [ei_ref[0].reshape(NG, 128), fill], axis=0).reshape(E_PAD)
    dst_ref[...] = jnp.concatenate(
        [ei_ref[1].reshape(NG, 128), fill], axis=0).reshape(E_PAD)


def _tc_mid(part_ref, degp_ref, r1_ref, bl1_ref, gamma_ref, beta_ref,
            wl2_ref, wr2_ref, p2_ref, r2_ref, invdeg_ref):
    acc = part_ref[0] + part_ref[1]
    deg = degp_ref[0, :, 0:1] + degp_ref[1, :, 0:1]
    invdeg = 1.0 / jnp.maximum(deg, 1.0)
    invdeg_ref[...] = invdeg
    h = acc * invdeg + bl1_ref[...] + r1_ref[...]
    rows = lax.broadcasted_iota(jnp.int32, (NP, 1), 0)
    mask = rows < N
    hm = jnp.where(mask, h, 0.0)
    mean = jnp.sum(hm, axis=0, keepdims=True) * (1.0 / N)
    cm = jnp.where(mask, h - mean, 0.0)
    var = jnp.sum(cm * cm, axis=0, keepdims=True) * (1.0 / N)
    hn = (h - mean) * lax.rsqrt(var + EPS) * gamma_ref[...] + beta_ref[...]
    hr = jnp.maximum(hn, 0.0)
    p2_ref[...] = lax.dot_general(hr, wl2_ref[...], _DN,
                                  preferred_element_type=jnp.float32)
    r2_ref[...] = lax.dot_general(hr, wr2_ref[...], _DN,
                                  preferred_element_type=jnp.float32)


def _tc_post(part2_ref, r2_ref, invdeg_ref, bl2_ref, out_ref):
    acc = part2_ref[0] + part2_ref[1]
    val = acc * invdeg_ref[...] + bl2_ref[...] + r2_ref[...]
    out_ref[...] = val[:N]


def kernel(x, edge_index, Wl1, bl1, Wr1, gamma, beta, Wl2, bl2, Wr2):
    ei = edge_index.astype(jnp.int32)
    z1 = jnp.zeros((ZCH, D1), jnp.float32)
    z2 = jnp.zeros((ZCH, D2), jnp.float32)
    zd = jnp.zeros((ZCH, DD), jnp.float32)
    onesb = jnp.zeros((128, DD), jnp.float32).at[:, 0].set(1.0)
    bl1r = bl1.reshape(1, HID)
    gammar = gamma.reshape(1, HID)
    betar = beta.reshape(1, HID)
    bl2r = bl2.reshape(1, OUT)

    p1ext, r1, src2d, dst2d = pl.pallas_call(
        _tc_pre,
        out_shape=[jax.ShapeDtypeStruct((NP, D1), jnp.float32),
                   jax.ShapeDtypeStruct((NP, HID), jnp.float32),
                   jax.ShapeDtypeStruct((E_PAD,), jnp.int32),
                   jax.ShapeDtypeStruct((E_PAD,), jnp.int32)],
    )(x, Wl1, Wr1, ei)

    part1, degp = _sc_agg1(p1ext, src2d, dst2d, z1, zd, onesb)

    p2, r2, invdeg = pl.pallas_call(
        _tc_mid,
        out_shape=[jax.ShapeDtypeStruct((NP, D2), jnp.float32),
                   jax.ShapeDtypeStruct((NP, D2), jnp.float32),
                   jax.ShapeDtypeStruct((NP, 1), jnp.float32)],
    )(part1, degp, r1, bl1r, gammar, betar, Wl2, Wr2)

    (part2,) = _sc_agg2(p2, src2d, dst2d, z2)

    out = pl.pallas_call(
        _tc_post,
        out_shape=jax.ShapeDtypeStruct((N, D2), jnp.float32),
    )(part2, r2, invdeg, bl2r)
    return out
